# Initial kernel scaffold; baseline (speedup 1.0000x reference)
import jax
import jax.numpy as jnp
from jax.experimental import pallas as pl


def kernel(x, edge_index1, W0, b0, W1, b1, W2, b2, Wp, bp):
    raise NotImplementedError("write your pallas kernel here")


# speedup vs baseline: 8.7371x; 8.7371x over previous
import functools
import jax
import jax.numpy as jnp
from jax import lax
from jax.experimental import pallas as pl
from jax.experimental.pallas import tpu as pltpu
from jax.experimental.pallas import tpu_sc as plsc

N = 10000
E = 320000
D_IN = 128
H2 = 256
HID = 128
PROJ = 64

NC = 2
NS = 16
CH = 128
ET = E + N
CPT = -(-ET // (NS * CH))
EPT = CPT * CH
E_PAD = NS * EPT
NPAD = 10240
RPT = NPAD // NS
DUMMY = N

RB = 400
GRID = N // RB

_mesh = plsc.VectorSubcoreMesh(
    core_axis_name="c", subcore_axis_name="s", num_cores=NC, num_subcores=NS
)


def _deg_body(dst_hbm, z_hbm, deg_hbm, idx, ones_b, acc):
    c = lax.axis_index("c")
    s = lax.axis_index("s")

    @pl.when(c == 0)
    def _():
        for k in range(CH // 16):
            ones_b[pl.ds(k * 16, 16)] = jnp.ones((16,), jnp.float32)
        pltpu.sync_copy(z_hbm.at[pl.ds(s * RPT, RPT)], acc.at[pl.ds(s * RPT, RPT)])
        plsc.subcore_barrier()

        def chunk(j, carry):
            off = s * EPT + j * CH
            pltpu.sync_copy(dst_hbm.at[pl.ds(off, CH)], idx)
            pltpu.sync_copy(ones_b, acc.at[idx], add=True)
            return carry

        lax.fori_loop(0, CPT, chunk, 0)
        plsc.subcore_barrier()
        pltpu.sync_copy(acc.at[pl.ds(s * RPT, RPT)], deg_hbm.at[pl.ds(s * RPT, RPT)])


def _deg_call(dst_pad, z1):
    f = pl.kernel(
        _deg_body,
        out_type=jax.ShapeDtypeStruct((NPAD,), jnp.float32),
        mesh=_mesh,
        scratch_types=[
            pltpu.VMEM((CH,), jnp.int32),
            pltpu.VMEM((CH,), jnp.float32),
            pltpu.VMEM_SHARED((NPAD,), jnp.float32),
        ],
    )
    return f(dst_pad, z1)


def _mp_body(hc, y0, y1, src_hbm, dst_hbm, z_hbm, s0_out, s1_out, si, di, rows, acc):
    c = lax.axis_index("c")
    s = lax.axis_index("s")

    pltpu.sync_copy(z_hbm.at[pl.ds(s * RPT, RPT)], acc.at[pl.ds(s * RPT, RPT)])
    plsc.subcore_barrier()

    def chunk(j, y_hbm):
        off = s * EPT + j * CH
        pltpu.sync_copy(src_hbm.at[pl.ds(off, CH)], si)
        pltpu.sync_copy(dst_hbm.at[pl.ds(off, CH)], di)
        pltpu.sync_copy(y_hbm.at[si], rows)
        pltpu.sync_copy(rows, acc.at[di], add=True)

    @pl.when(c == 0)
    def _():
        lax.fori_loop(0, CPT, lambda j, k: (chunk(j, y0), k)[1], 0)

    @pl.when(c == 1)
    def _():
        lax.fori_loop(0, CPT, lambda j, k: (chunk(j, y1), k)[1], 0)

    plsc.subcore_barrier()

    @pl.when(c == 0)
    def _():
        pltpu.sync_copy(acc.at[pl.ds(s * RPT, RPT)], s0_out.at[pl.ds(s * RPT, RPT)])

    @pl.when(c == 1)
    def _():
        pltpu.sync_copy(acc.at[pl.ds(s * RPT, RPT)], s1_out.at[pl.ds(s * RPT, RPT)])


def _mp_call(hc, y0, y1, src_pad, dst_pad, z2):
    f = pl.kernel(
        functools.partial(_mp_body, hc),
        out_type=[jax.ShapeDtypeStruct((NPAD, hc), jnp.float32)] * 2,
        mesh=_mesh,
        scratch_types=[
            pltpu.VMEM((CH,), jnp.int32),
            pltpu.VMEM((CH,), jnp.int32),
            pltpu.VMEM((CH, hc), jnp.float32),
            pltpu.VMEM_SHARED((NPAD, hc), jnp.float32),
        ],
    )
    return f(y0, y1, src_pad, dst_pad, z2)


CPT2 = CPT // 2


def _mp_edge_body(hc, y, src_hbm, dst_hbm, z_hbm, s0_out, s1_out, si, di, rows, acc):
    c = lax.axis_index("c")
    s = lax.axis_index("s")

    pltpu.sync_copy(z_hbm.at[pl.ds(s * RPT, RPT)], acc.at[pl.ds(s * RPT, RPT)])
    plsc.subcore_barrier()

    def chunk(j, _):
        off = (c * NS + s) * (CPT2 * CH) + j * CH
        pltpu.sync_copy(src_hbm.at[pl.ds(off, CH)], si)
        pltpu.sync_copy(dst_hbm.at[pl.ds(off, CH)], di)
        pltpu.sync_copy(y.at[si], rows)
        pltpu.sync_copy(rows, acc.at[di], add=True)
        return 0

    lax.fori_loop(0, CPT2, chunk, 0)
    plsc.subcore_barrier()

    @pl.when(c == 0)
    def _():
        pltpu.sync_copy(acc.at[pl.ds(s * RPT, RPT)], s0_out.at[pl.ds(s * RPT, RPT)])

    @pl.when(c == 1)
    def _():
        pltpu.sync_copy(acc.at[pl.ds(s * RPT, RPT)], s1_out.at[pl.ds(s * RPT, RPT)])


def _mp_edge_call(hc, y, src_pad, dst_pad, z):
    f = pl.kernel(
        functools.partial(_mp_edge_body, hc),
        out_type=[jax.ShapeDtypeStruct((NPAD, hc), jnp.float32)] * 2,
        mesh=_mesh,
        scratch_types=[
            pltpu.VMEM((CH,), jnp.int32),
            pltpu.VMEM((CH,), jnp.int32),
            pltpu.VMEM((CH, hc), jnp.float32),
            pltpu.VMEM_SHARED((NPAD, hc), jnp.float32),
        ],
    )
    return f(y, src_pad, dst_pad, z)


def _dis(deg_ref):
    return lax.rsqrt(jnp.maximum(deg_ref[...], 1.0))


def _lin1_body(x_ref, w_ref, deg_ref, y0_ref, y1_ref):
    dis = _dis(deg_ref)
    y = jnp.dot(x_ref[...], w_ref[...], preferred_element_type=jnp.float32) * dis
    y0_ref[...] = y[:, : H2 // 2]
    y1_ref[...] = y[:, H2 // 2 :]


def _lin1_call(x, w0, deg2):
    return pl.pallas_call(
        _lin1_body,
        grid=(GRID,),
        in_specs=[
            pl.BlockSpec((RB, D_IN), lambda i: (i, 0)),
            pl.BlockSpec((D_IN, H2), lambda i: (0, 0)),
            pl.BlockSpec((RB, 1), lambda i: (i, 0)),
        ],
        out_specs=[
            pl.BlockSpec((RB, H2 // 2), lambda i: (i, 0)),
            pl.BlockSpec((RB, H2 // 2), lambda i: (i, 0)),
        ],
        out_shape=[jax.ShapeDtypeStruct((N, H2 // 2), jnp.float32)] * 2,
    )(x, w0, deg2)


def _mid_body(split_out, s0_ref, s1_ref, deg_ref, w_ref, b_ref, *out_refs):
    dis = _dis(deg_ref)
    h = jnp.concatenate([s0_ref[...], s1_ref[...]], axis=1)
    h = jax.nn.relu(dis * h + b_ref[...])
    y = jnp.dot(h, w_ref[...], preferred_element_type=jnp.float32) * dis
    if split_out:
        hh = w_ref.shape[1] // 2
        out_refs[0][...] = y[:, :hh]
        out_refs[1][...] = y[:, hh:]
    else:
        out_refs[0][...] = y


def _mid_call(s0, s1, deg2, w, b2d, split_out=True):
    hin = w.shape[0]
    hout = w.shape[1]
    if split_out:
        out_specs = [
            pl.BlockSpec((RB, hout // 2), lambda i: (i, 0)),
            pl.BlockSpec((RB, hout // 2), lambda i: (i, 0)),
        ]
        out_shape = [jax.ShapeDtypeStruct((N, hout // 2), jnp.float32)] * 2
    else:
        out_specs = pl.BlockSpec((RB, hout), lambda i: (i, 0))
        out_shape = jax.ShapeDtypeStruct((N, hout), jnp.float32)
    return pl.pallas_call(
        functools.partial(_mid_body, split_out),
        grid=(GRID,),
        in_specs=[
            pl.BlockSpec((RB, hin // 2), lambda i: (i, 0)),
            pl.BlockSpec((RB, hin // 2), lambda i: (i, 0)),
            pl.BlockSpec((RB, 1), lambda i: (i, 0)),
            pl.BlockSpec((hin, hout), lambda i: (0, 0)),
            pl.BlockSpec((1, hin), lambda i: (0, 0)),
        ],
        out_specs=out_specs,
        out_shape=out_shape,
    )(s0, s1, deg2, w, b2d)


def _fin_body(s0_ref, s1_ref, deg_ref, b2_ref, wp_ref, bp_ref, out_ref):
    dis = _dis(deg_ref)
    h = s0_ref[...] + s1_ref[...]
    h = jax.nn.relu(dis * h + b2_ref[...])
    p = jax.nn.relu(
        jnp.dot(h, wp_ref[...], preferred_element_type=jnp.float32) + bp_ref[...]
    )
    nrm = jnp.sqrt(jnp.sum(p * p, axis=1, keepdims=True))
    out_ref[...] = p / jnp.maximum(nrm, 1e-12)


def _fin_call(s0, s1, deg2, b2d, wp, bp2d):
    return pl.pallas_call(
        _fin_body,
        grid=(GRID,),
        in_specs=[
            pl.BlockSpec((RB, HID), lambda i: (i, 0)),
            pl.BlockSpec((RB, HID), lambda i: (i, 0)),
            pl.BlockSpec((RB, 1), lambda i: (i, 0)),
            pl.BlockSpec((1, HID), lambda i: (0, 0)),
            pl.BlockSpec((HID, PROJ), lambda i: (0, 0)),
            pl.BlockSpec((1, PROJ), lambda i: (0, 0)),
        ],
        out_specs=pl.BlockSpec((RB, PROJ), lambda i: (i, 0)),
        out_shape=jax.ShapeDtypeStruct((N, PROJ), jnp.float32),
    )(s0, s1, deg2, b2d, wp, bp2d)


def kernel(x, edge_index1, W0, b0, W1, b1, W2, b2, Wp, bp):
    loop = jnp.arange(N, dtype=jnp.int32)
    pad = E_PAD - ET
    src_pad = jnp.concatenate(
        [edge_index1[0], loop, jnp.zeros((pad,), jnp.int32)]
    )
    dst_pad = jnp.concatenate(
        [edge_index1[1], loop, jnp.full((pad,), DUMMY, jnp.int32)]
    )
    z1 = jnp.zeros((NPAD,), jnp.float32)
    z2 = jnp.zeros((NPAD, H2 // 2), jnp.float32)

    deg = _deg_call(dst_pad, z1)
    deg2 = deg[:, None]

    y0a, y0b = _lin1_call(x, W0, deg2)
    s1a, s1b = _mp_call(H2 // 2, y0a, y0b, src_pad, dst_pad, z2)

    y1a, y1b = _mid_call(s1a, s1b, deg2, W1, b0[None, :])
    s2a, s2b = _mp_call(H2 // 2, y1a, y1b, src_pad, dst_pad, z2)

    y2 = _mid_call(s2a, s2b, deg2, W2, b1[None, :], split_out=False)
    s3a, s3b = _mp_edge_call(HID, y2, src_pad, dst_pad, z2)

    return _fin_call(s3a, s3b, deg2, b2[None, :], Wp, bp[None, :])
